# Initial kernel scaffold; baseline (speedup 1.0000x reference)
import jax
import jax.numpy as jnp
from jax.experimental import pallas as pl


def kernel(x, edge_index, W1, W2):
    raise NotImplementedError("write your pallas kernel here")


# speedup vs baseline: 11.9272x; 11.9272x over previous
import functools

import jax
import jax.numpy as jnp
from jax import lax
from jax.experimental import pallas as pl
from jax.experimental.pallas import tpu as pltpu
from jax.experimental.pallas import tpu_sc as plsc

_N = 10000
_E = 320000
_NC = 2
_NS = 16
_NW = _NC * _NS
_CH = 128
_EPT = 10112
_E_PAD = _EPT * _NW
_NPAD = 10240
_RPT = _NPAD // _NS

_mesh = plsc.VectorSubcoreMesh(core_axis_name="c", subcore_axis_name="s")


def _make_scatter(D):
  @functools.partial(
      pl.kernel,
      out_type=jax.ShapeDtypeStruct((_NC * _NPAD, D), jnp.float32),
      mesh=_mesh,
      compiler_params=pltpu.CompilerParams(use_tc_tiling_on_sc=(D == 128)),
      scratch_types=[
          pltpu.VMEM((_CH,), jnp.int32),
          pltpu.VMEM((_CH,), jnp.int32),
          pltpu.VMEM((_CH, D), jnp.float32),
          pltpu.VMEM_SHARED((_NPAD, D), jnp.float32),
          pltpu.SemaphoreType.DMA,
      ],
  )
  def scat(y_hbm, src_hbm, dst_hbm, out_hbm, src_v, dst_v, rows_v, z_sh, sem):
    cid = lax.axis_index("c")
    sid = lax.axis_index("s")
    wid = sid * _NC + cid

    def zrow(r, _):
      for c in range(D // 16):
        rows_v[r, pl.ds(c * 16, 16)] = jnp.zeros((16,), jnp.float32)
      return 0
    lax.fori_loop(0, _CH, zrow, 0)
    for j in range(_RPT // _CH):
      pltpu.sync_copy(rows_v, z_sh.at[pl.ds(sid * _RPT + j * _CH, _CH)])
    plsc.subcore_barrier()

    base = wid * _EPT
    def body(k, _):
      off = base + k * _CH
      pltpu.sync_copy(src_hbm.at[pl.ds(off, _CH)], src_v)
      pltpu.sync_copy(dst_hbm.at[pl.ds(off, _CH)], dst_v)
      pltpu.async_copy(y_hbm.at[src_v], rows_v, sem).wait()
      pltpu.sync_copy(rows_v, z_sh.at[dst_v], add=True)
      return 0
    lax.fori_loop(0, _EPT // _CH, body, 0)
    plsc.subcore_barrier()

    pltpu.sync_copy(z_sh.at[pl.ds(sid * _RPT, _RPT)],
                    out_hbm.at[pl.ds(cid * _NPAD + sid * _RPT, _RPT)])

  return scat


_scat128 = _make_scatter(128)
_scat64 = _make_scatter(64)


@functools.partial(
    pl.kernel,
    out_type=jax.ShapeDtypeStruct((_NC * _NPAD,), jnp.float32),
    mesh=_mesh,
    scratch_types=[
        pltpu.VMEM((_CH,), jnp.int32),
        pltpu.VMEM((_CH,), jnp.float32),
        pltpu.VMEM((_RPT,), jnp.float32),
        pltpu.VMEM_SHARED((_NPAD,), jnp.float32),
    ],
)
def _deg_kernel(dst_hbm, out_hbm, dst_v, ones_v, zer_v, d_sh):
  cid = lax.axis_index("c")
  sid = lax.axis_index("s")
  wid = sid * _NC + cid

  for c in range(_CH // 16):
    ones_v[pl.ds(c * 16, 16)] = jnp.ones((16,), jnp.float32)
  for c in range(_RPT // 16):
    zer_v[pl.ds(c * 16, 16)] = jnp.zeros((16,), jnp.float32)
  pltpu.sync_copy(zer_v, d_sh.at[pl.ds(sid * _RPT, _RPT)])
  plsc.subcore_barrier()

  base = wid * _EPT
  def body(k, _):
    off = base + k * _CH
    pltpu.sync_copy(dst_hbm.at[pl.ds(off, _CH)], dst_v)
    pltpu.sync_copy(ones_v, d_sh.at[dst_v], add=True)
    return 0
  lax.fori_loop(0, _EPT // _CH, body, 0)
  plsc.subcore_barrier()

  pltpu.sync_copy(d_sh.at[pl.ds(sid * _RPT, _RPT)],
                  out_hbm.at[pl.ds(cid * _NPAD + sid * _RPT, _RPT)])


_BLK = 1000


def _tc1_body(x_ref, w_ref, d0_ref, d1_ref, y_ref, dinv_ref):
  deg = d0_ref[...] + d1_ref[...] + 1.0
  dinv = lax.rsqrt(deg)
  dinv_ref[...] = dinv
  y_ref[...] = jnp.dot(x_ref[...], w_ref[...],
                       preferred_element_type=jnp.float32) * dinv


def _tc1(x, W1, d0, d1):
  return pl.pallas_call(
      _tc1_body,
      grid=(_N // _BLK,),
      in_specs=[
          pl.BlockSpec((_BLK, 128), lambda i: (i, 0)),
          pl.BlockSpec((128, 128), lambda i: (0, 0)),
          pl.BlockSpec((_BLK, 1), lambda i: (i, 0)),
          pl.BlockSpec((_BLK, 1), lambda i: (i, 0)),
      ],
      out_specs=[
          pl.BlockSpec((_BLK, 128), lambda i: (i, 0)),
          pl.BlockSpec((_BLK, 1), lambda i: (i, 0)),
      ],
      out_shape=[
          jax.ShapeDtypeStruct((_N, 128), jnp.float32),
          jax.ShapeDtypeStruct((_N, 1), jnp.float32),
      ],
  )(x, W1, d0, d1)


def _tc2_body(y_ref, sa_ref, sb_ref, dinv_ref, w_ref, out_ref):
  dinv = dinv_ref[...]
  z = (y_ref[...] + sa_ref[...] + sb_ref[...]) * dinv
  h = jnp.maximum(z, 0.0)
  out_ref[...] = jnp.dot(h, w_ref[...],
                         preferred_element_type=jnp.float32) * dinv


def _tc2(y1, s1a, s1b, dinv, W2):
  return pl.pallas_call(
      _tc2_body,
      grid=(_N // _BLK,),
      in_specs=[
          pl.BlockSpec((_BLK, 128), lambda i: (i, 0)),
          pl.BlockSpec((_BLK, 128), lambda i: (i, 0)),
          pl.BlockSpec((_BLK, 128), lambda i: (i, 0)),
          pl.BlockSpec((_BLK, 1), lambda i: (i, 0)),
          pl.BlockSpec((128, 64), lambda i: (0, 0)),
      ],
      out_specs=pl.BlockSpec((_BLK, 64), lambda i: (i, 0)),
      out_shape=jax.ShapeDtypeStruct((_N, 64), jnp.float32),
  )(y1, s1a, s1b, dinv, W2)


def _tc3_body(y_ref, sa_ref, sb_ref, dinv_ref, out_ref):
  z = (y_ref[...] + sa_ref[...] + sb_ref[...]) * dinv_ref[...]
  out_ref[...] = jnp.maximum(z, 0.0)


def _tc3(y2, s2a, s2b, dinv):
  return pl.pallas_call(
      _tc3_body,
      grid=(_N // _BLK,),
      in_specs=[
          pl.BlockSpec((_BLK, 64), lambda i: (i, 0)),
          pl.BlockSpec((_BLK, 64), lambda i: (i, 0)),
          pl.BlockSpec((_BLK, 64), lambda i: (i, 0)),
          pl.BlockSpec((_BLK, 1), lambda i: (i, 0)),
      ],
      out_specs=pl.BlockSpec((_BLK, 64), lambda i: (i, 0)),
      out_shape=jax.ShapeDtypeStruct((_N, 64), jnp.float32),
  )(y2, s2a, s2b, dinv)


def kernel(x, edge_index, W1, W2):
  pad = _E_PAD - _E
  src_p = jnp.concatenate([edge_index[0], jnp.zeros((pad,), jnp.int32)])
  dst_p = jnp.concatenate([edge_index[1], jnp.full((pad,), _N, jnp.int32)])

  deg2 = _deg_kernel(dst_p)
  d0 = deg2[:_N].reshape(_N, 1)
  d1 = deg2[_NPAD:_NPAD + _N].reshape(_N, 1)

  y1, dinv = _tc1(x, W1, d0, d1)
  s1 = _scat128(y1, src_p, dst_p)
  y2 = _tc2(y1, s1[:_N], s1[_NPAD:_NPAD + _N], dinv, W2)
  s2 = _scat64(y2, src_p, dst_p)
  return _tc3(y2, s2[:_N], s2[_NPAD:_NPAD + _N], dinv)


# speedup vs baseline: 12.4531x; 1.0441x over previous
import functools

import jax
import jax.numpy as jnp
from jax import lax
from jax.experimental import pallas as pl
from jax.experimental.pallas import tpu as pltpu
from jax.experimental.pallas import tpu_sc as plsc

_N = 10000
_E = 320000
_NC = 2
_NS = 16
_CH = 128
_KT = 160
_EPT = _KT * _CH
_E_PAD = _EPT * _NS
_NCHUNK = _E_PAD // _CH
_NPAD = 10240
_RPT = _NPAD // _NS

_mesh = plsc.VectorSubcoreMesh(core_axis_name="c", subcore_axis_name="s")


def _make_scatter(Dh):
  @functools.partial(
      pl.kernel,
      out_type=jax.ShapeDtypeStruct((_NC * _NPAD, Dh), jnp.float32),
      mesh=_mesh,
      compiler_params=pltpu.CompilerParams(use_tc_tiling_on_sc=False),
      scratch_types=[
          pltpu.VMEM((_KT, _CH), jnp.int32),
          pltpu.VMEM((_KT, _CH), jnp.int32),
          pltpu.VMEM((_CH, Dh), jnp.float32),
          pltpu.VMEM((_CH, Dh), jnp.float32),
          pltpu.VMEM_SHARED((_NPAD, Dh), jnp.float32),
          pltpu.SemaphoreType.DMA,
          pltpu.SemaphoreType.DMA,
          pltpu.SemaphoreType.DMA,
          pltpu.SemaphoreType.DMA,
      ],
  )
  def scat(y_hbm, srcb_hbm, dst_hbm, out_hbm, src_v, dst_v,
           buf0, buf1, z_sh, g0, g1, s0, s1):
    cid = lax.axis_index("c")
    sid = lax.axis_index("s")
    bufs = (buf0, buf1)
    gs = (g0, g1)
    ss = (s0, s1)

    pltpu.sync_copy(srcb_hbm.at[cid, pl.ds(sid * _KT, _KT)], src_v)
    pltpu.sync_copy(dst_hbm.at[pl.ds(sid * _KT, _KT)], dst_v)

    def zrow(r, _):
      for c in range(Dh // 16):
        buf0[r, pl.ds(c * 16, 16)] = jnp.zeros((16,), jnp.float32)
      return 0
    lax.fori_loop(0, _CH, zrow, 0)
    for j in range(_RPT // _CH):
      pltpu.sync_copy(buf0, z_sh.at[pl.ds(sid * _RPT + j * _CH, _CH)])
    plsc.subcore_barrier()

    def g_start(k, b):
      pltpu.async_copy(y_hbm.at[src_v.at[k]], bufs[b], gs[b])

    def g_wait(k, b):
      pltpu.make_async_copy(y_hbm.at[src_v.at[k]], bufs[b], gs[b]).wait()

    def s_start(k, b):
      pltpu.async_copy(bufs[b], z_sh.at[dst_v.at[k]], ss[b], add=True)

    def s_wait(k, b):
      pltpu.make_async_copy(bufs[b], z_sh.at[dst_v.at[k]], ss[b]).wait()

    def step(k, b, do_swait, do_gstart):
      g_wait(k, b)
      s_start(k, b)
      if do_swait:
        s_wait(k - 1, 1 - b)
      if do_gstart:
        g_start(k + 1, 1 - b)

    g_start(0, 0)
    step(0, 0, False, True)
    step(1, 1, True, True)

    def body(i, _):
      k = i * 2
      step(k, 0, True, True)
      step(k + 1, 1, True, True)
      return 0
    lax.fori_loop(1, _KT // 2 - 1, body, 0)

    step(_KT - 2, 0, True, True)
    step(_KT - 1, 1, True, False)
    s_wait(_KT - 1, 1)

    plsc.subcore_barrier()
    pltpu.sync_copy(z_sh.at[pl.ds(sid * _RPT, _RPT)],
                    out_hbm.at[pl.ds(cid * _NPAD + sid * _RPT, _RPT)])

  return scat


_scat64 = _make_scatter(64)
_scat32 = _make_scatter(32)


@functools.partial(
    pl.kernel,
    out_type=jax.ShapeDtypeStruct((_NC * _NPAD,), jnp.float32),
    mesh=_mesh,
    scratch_types=[
        pltpu.VMEM((_KT // 2, _CH), jnp.int32),
        pltpu.VMEM((_CH,), jnp.float32),
        pltpu.VMEM((_RPT,), jnp.float32),
        pltpu.VMEM_SHARED((_NPAD,), jnp.float32),
        pltpu.SemaphoreType.DMA,
        pltpu.SemaphoreType.DMA,
    ],
)
def _deg_kernel(dst_hbm, out_hbm, dst_v, ones_v, zer_v, d_sh, s0, s1):
  cid = lax.axis_index("c")
  sid = lax.axis_index("s")
  wid = sid * _NC + cid
  kw = _KT // 2
  ss = (s0, s1)

  pltpu.sync_copy(dst_hbm.at[pl.ds(wid * kw, kw)], dst_v)
  for c in range(_CH // 16):
    ones_v[pl.ds(c * 16, 16)] = jnp.ones((16,), jnp.float32)
  for c in range(_RPT // 16):
    zer_v[pl.ds(c * 16, 16)] = jnp.zeros((16,), jnp.float32)
  pltpu.sync_copy(zer_v, d_sh.at[pl.ds(sid * _RPT, _RPT)])
  plsc.subcore_barrier()

  def s_start(k, b):
    pltpu.async_copy(ones_v, d_sh.at[dst_v.at[k]], ss[b], add=True)

  def s_wait(k, b):
    pltpu.make_async_copy(ones_v, d_sh.at[dst_v.at[k]], ss[b]).wait()

  s_start(0, 0)
  s_start(1, 1)
  def body(i, _):
    k = i * 2
    s_wait(k - 2, 0)
    s_start(k, 0)
    s_wait(k - 1, 1)
    s_start(k + 1, 1)
    return 0
  lax.fori_loop(1, kw // 2, body, 0)
  s_wait(kw - 2, 0)
  s_wait(kw - 1, 1)

  plsc.subcore_barrier()
  pltpu.sync_copy(d_sh.at[pl.ds(sid * _RPT, _RPT)],
                  out_hbm.at[pl.ds(cid * _NPAD + sid * _RPT, _RPT)])


_BLK = 1000


def _tc1_body(x_ref, w_ref, d0_ref, d1_ref, y_ref, dinv_ref):
  deg = d0_ref[...] + d1_ref[...] + 1.0
  dinv = lax.rsqrt(deg)
  dinv_ref[...] = dinv
  y_ref[...] = jnp.dot(x_ref[...], w_ref[...],
                       preferred_element_type=jnp.float32) * dinv


def _tc1(x, W1, d0, d1):
  return pl.pallas_call(
      _tc1_body,
      grid=(_N // _BLK,),
      in_specs=[
          pl.BlockSpec((_BLK, 128), lambda i: (i, 0)),
          pl.BlockSpec((128, 128), lambda i: (0, 0)),
          pl.BlockSpec((_BLK, 1), lambda i: (i, 0)),
          pl.BlockSpec((_BLK, 1), lambda i: (i, 0)),
      ],
      out_specs=[
          pl.BlockSpec((_BLK, 128), lambda i: (i, 0)),
          pl.BlockSpec((_BLK, 1), lambda i: (i, 0)),
      ],
      out_shape=[
          jax.ShapeDtypeStruct((_N, 128), jnp.float32),
          jax.ShapeDtypeStruct((_N, 1), jnp.float32),
      ],
  )(x, W1, d0, d1)


def _tc2_body(y_ref, sa_ref, sb_ref, dinv_ref, w_ref, out_ref):
  dinv = dinv_ref[...]
  s = jnp.concatenate([sa_ref[...], sb_ref[...]], axis=1)
  z = (y_ref[...] + s) * dinv
  h = jnp.maximum(z, 0.0)
  out_ref[...] = jnp.dot(h, w_ref[...],
                         preferred_element_type=jnp.float32) * dinv


def _tc2(y1, s1a, s1b, dinv, W2):
  return pl.pallas_call(
      _tc2_body,
      grid=(_N // _BLK,),
      in_specs=[
          pl.BlockSpec((_BLK, 128), lambda i: (i, 0)),
          pl.BlockSpec((_BLK, 64), lambda i: (i, 0)),
          pl.BlockSpec((_BLK, 64), lambda i: (i, 0)),
          pl.BlockSpec((_BLK, 1), lambda i: (i, 0)),
          pl.BlockSpec((128, 64), lambda i: (0, 0)),
      ],
      out_specs=pl.BlockSpec((_BLK, 64), lambda i: (i, 0)),
      out_shape=jax.ShapeDtypeStruct((_N, 64), jnp.float32),
  )(y1, s1a, s1b, dinv, W2)


def _tc3_body(y_ref, sa_ref, sb_ref, dinv_ref, out_ref):
  s = jnp.concatenate([sa_ref[...], sb_ref[...]], axis=1)
  z = (y_ref[...] + s) * dinv_ref[...]
  out_ref[...] = jnp.maximum(z, 0.0)


def _tc3(y2, s2a, s2b, dinv):
  return pl.pallas_call(
      _tc3_body,
      grid=(_N // _BLK,),
      in_specs=[
          pl.BlockSpec((_BLK, 64), lambda i: (i, 0)),
          pl.BlockSpec((_BLK, 32), lambda i: (i, 0)),
          pl.BlockSpec((_BLK, 32), lambda i: (i, 0)),
          pl.BlockSpec((_BLK, 1), lambda i: (i, 0)),
      ],
      out_specs=pl.BlockSpec((_BLK, 64), lambda i: (i, 0)),
      out_shape=jax.ShapeDtypeStruct((_N, 64), jnp.float32),
  )(y2, s2a, s2b, dinv)


def kernel(x, edge_index, W1, W2):
  pad = _E_PAD - _E
  src_p = jnp.concatenate([edge_index[0], jnp.zeros((pad,), jnp.int32)])
  dst_p = jnp.concatenate(
      [edge_index[1], jnp.full((pad,), _N, jnp.int32)]).reshape(_NCHUNK, _CH)
  srcb = jnp.stack([2 * src_p, 2 * src_p + 1]).reshape(_NC, _NCHUNK, _CH)

  deg2 = _deg_kernel(dst_p)
  d0 = deg2[:_N].reshape(_N, 1)
  d1 = deg2[_NPAD:_NPAD + _N].reshape(_N, 1)

  y1, dinv = _tc1(x, W1, d0, d1)
  s1 = _scat64(y1.reshape(2 * _N, 64), srcb, dst_p)
  y2 = _tc2(y1, s1[:_N], s1[_NPAD:_NPAD + _N], dinv, W2)
  s2 = _scat32(y2.reshape(2 * _N, 32), srcb, dst_p)
  return _tc3(y2, s2[:_N], s2[_NPAD:_NPAD + _N], dinv)
